# baseline (speedup 1.0000x reference)
import functools

import jax
import jax.numpy as jnp
from jax import lax
from jax.experimental import pallas as pl
from jax.experimental.pallas import tpu as pltpu
from jax.experimental.pallas import tpu_sc as plsc

_NEG = -1e30


def _sc_topk_kernel(masks_hbm, out_hbm, in_v, out_v):
    b = lax.axis_index("s") * 2 + lax.axis_index("c")
    pltpu.sync_copy(masks_hbm.at[b], in_v)
    slot = lax.iota(jnp.int32, 16)
    sign = jnp.int32(-2147483648)
    n_pos = out_v.shape[0] // 16
    row = slot * n_pos

    def body(p, carry):
        idx = row + p
        v = plsc.load_gather(in_v, [idx])
        bits = plsc.bitcast(v, jnp.int32)
        key = ((bits << 4) | (15 - slot)) ^ sign
        sk = lax.sort(key)
        thr = jnp.max(jnp.where(slot == 8, sk, sign))
        vals = jnp.where(key >= thr, 0.0, _NEG).astype(jnp.float32)
        plsc.store_scatter(out_v, [idx], vals)
        return carry

    lax.fori_loop(0, n_pos, body, 0)
    pltpu.sync_copy(out_v, out_hbm.at[b])


def _prep_kernel(x_ref, w1_ref, b1_ref, w2_ref, feat_ref, pos_ref,
                 w2d_ref, wa_ref):
    n_feat = feat_ref.shape[0]
    out_dim = w2d_ref.shape[1]
    y = jax.lax.dot(x_ref[...], w1_ref[...], precision=jax.lax.Precision.HIGHEST)
    feat_ref[...] = y[:n_feat].astype(jnp.bfloat16)
    pos_ref[...] = (y[n_feat:] + b1_ref[...]).astype(jnp.bfloat16)
    w2d_ref[...] = w2_ref[:, :out_dim].astype(jnp.bfloat16)
    wa_ref[...] = w2_ref[:, w2_ref.shape[1] - 128:].astype(jnp.bfloat16)


def _decode_kernel(feat_ref, sn_ref, pos_ref, w2_ref, wa_ref, b2_ref, out_ref):
    k_slots = feat_ref.shape[1]
    feat = feat_ref[0]
    sn = sn_ref[0]
    pos = pos_ref[...]

    x = feat[:, None, :] + pos[None, :, :]
    c0 = jnp.bfloat16(0.7978845608028654)
    c1 = jnp.bfloat16(0.7978845608028654 * 0.044715)
    t = x * x
    y = x * (c0 + c1 * t)
    r = jnp.bfloat16(0.5) * x
    h_all = r + r * jnp.tanh(y)

    p_dim, h_dim = pos.shape
    a3 = jax.lax.dot(h_all.reshape(k_slots * p_dim, h_dim), wa_ref[...],
                     preferred_element_type=jnp.float32)
    alpha = a3[:, 127].reshape(k_slots, p_dim)

    a = alpha + sn
    amax = jnp.max(a, axis=0, keepdims=True)
    e = jnp.exp(a - amax)
    w = e / jnp.sum(e, axis=0, keepdims=True)

    wb = w.astype(jnp.bfloat16)
    hw = h_all * wb[:, :, None]
    terms = [hw[k] for k in range(k_slots)]
    while len(terms) > 1:
        terms = [terms[i] + terms[i + 1] for i in range(0, len(terms), 2)]
    hbar = terms[0]

    out_ref[0] = jax.lax.dot(
        hbar, w2_ref[...],
        preferred_element_type=jnp.float32) + b2_ref[...]


def kernel(object_features, masks, pos_embed, W1, b1, W2, b2):
    b, k_slots, d = object_features.shape
    p = pos_embed.shape[1]
    h = W1.shape[1]
    out_dim = W2.shape[1] - 1

    sc_topk = functools.partial(
        pl.kernel,
        out_type=jax.ShapeDtypeStruct((b, k_slots * p), jnp.float32),
        mesh=plsc.VectorSubcoreMesh(core_axis_name="c", subcore_axis_name="s"),
        scratch_types=[
            pltpu.VMEM((k_slots * p,), jnp.float32),
            pltpu.VMEM((k_slots * p,), jnp.float32),
        ],
        compiler_params=pltpu.CompilerParams(needs_layout_passes=False),
    )(_sc_topk_kernel)
    selneg = sc_topk(masks.reshape(b, k_slots * p)).reshape(b, k_slots, p)

    x = jnp.concatenate([object_features.reshape(b * k_slots, d), pos_embed[0]], axis=0)
    feat_proj, pos_base, w2_dec, w_alpha = pl.pallas_call(
        _prep_kernel,
        out_shape=(
            jax.ShapeDtypeStruct((b * k_slots, h), jnp.bfloat16),
            jax.ShapeDtypeStruct((p, h), jnp.bfloat16),
            jax.ShapeDtypeStruct((h, out_dim), jnp.bfloat16),
            jax.ShapeDtypeStruct((h, 128), jnp.bfloat16),
        ),
    )(x, W1, b1.reshape(1, h), W2)

    feat_proj = feat_proj.reshape(b, k_slots, h)
    b2_dec = b2[:out_dim].reshape(1, out_dim)

    out = pl.pallas_call(
        _decode_kernel,
        grid=(b,),
        in_specs=[
            pl.BlockSpec((1, k_slots, h), lambda i: (i, 0, 0)),
            pl.BlockSpec((1, k_slots, p), lambda i: (i, 0, 0)),
            pl.BlockSpec((p, h), lambda i: (0, 0)),
            pl.BlockSpec((h, out_dim), lambda i: (0, 0)),
            pl.BlockSpec((h, 128), lambda i: (0, 0)),
            pl.BlockSpec((1, out_dim), lambda i: (0, 0)),
        ],
        out_specs=pl.BlockSpec((1, p, out_dim), lambda i: (i, 0, 0)),
        out_shape=jax.ShapeDtypeStruct((b, p, out_dim), jnp.float32),
    )(feat_proj, selneg, pos_base, w2_dec, w_alpha, b2_dec)
    return out


# speedup vs baseline: 1.0169x; 1.0169x over previous
import functools

import jax
import jax.numpy as jnp
from jax import lax
from jax.experimental import pallas as pl
from jax.experimental.pallas import tpu as pltpu
from jax.experimental.pallas import tpu_sc as plsc

_NEG = -1e30


def _sc_topk_kernel(masks_hbm, out_hbm, in_v, out_v):
    b = lax.axis_index("s") * 2 + lax.axis_index("c")
    pltpu.sync_copy(masks_hbm.at[b], in_v)
    k_slots = 16
    n_pos = out_v.shape[0] // k_slots

    def body(g, carry):
        base = g * 16
        keys = []
        for k in range(k_slots):
            v = in_v[pl.ds(k * n_pos + base, 16)]
            bits = plsc.bitcast(v, jnp.int32)
            keys.append((bits << 4) | (15 - k))
        for k in range(k_slots):
            rank = (keys[0] > keys[k]).astype(jnp.int32)
            for j in range(1, k_slots):
                rank = rank + (keys[j] > keys[k]).astype(jnp.int32)
            vals = jnp.where(rank < 8, 0.0, _NEG).astype(jnp.float32)
            out_v[pl.ds(k * n_pos + base, 16)] = vals
        return carry

    lax.fori_loop(0, n_pos // 16, body, 0)
    pltpu.sync_copy(out_v, out_hbm.at[b])


def _prep_kernel(x_ref, w1_ref, b1_ref, w2_ref, feat_ref, pos_ref,
                 w2d_ref, wa_ref):
    n_feat = feat_ref.shape[0]
    out_dim = w2d_ref.shape[1]
    y = jax.lax.dot(x_ref[...], w1_ref[...], precision=jax.lax.Precision.HIGHEST)
    feat_ref[...] = y[:n_feat].astype(jnp.bfloat16)
    pos_ref[...] = (y[n_feat:] + b1_ref[...]).astype(jnp.bfloat16)
    w2d_ref[...] = w2_ref[:, :out_dim].astype(jnp.bfloat16)
    wa_ref[...] = w2_ref[:, w2_ref.shape[1] - 128:].astype(jnp.bfloat16)


def _decode_kernel(feat_ref, sn_ref, pos_ref, w2_ref, wa_ref, b2_ref, out_ref):
    k_slots = feat_ref.shape[1]
    feat = feat_ref[0]
    sn = sn_ref[0]
    pos = pos_ref[...]

    x = feat[:, None, :] + pos[None, :, :]
    c0 = jnp.bfloat16(0.7978845608028654)
    c1 = jnp.bfloat16(0.7978845608028654 * 0.044715)
    t = x * x
    y = x * (c0 + c1 * t)
    r = jnp.bfloat16(0.5) * x
    h_all = r + r * jnp.tanh(y)

    p_dim, h_dim = pos.shape
    a3 = jax.lax.dot(h_all.reshape(k_slots * p_dim, h_dim), wa_ref[...],
                     preferred_element_type=jnp.float32)
    alpha = a3[:, 127].reshape(k_slots, p_dim)

    a = alpha + sn
    amax = jnp.max(a, axis=0, keepdims=True)
    e = jnp.exp(a - amax)
    w = e / jnp.sum(e, axis=0, keepdims=True)

    wb = w.astype(jnp.bfloat16)
    hw = h_all * wb[:, :, None]
    terms = [hw[k] for k in range(k_slots)]
    while len(terms) > 1:
        terms = [terms[i] + terms[i + 1] for i in range(0, len(terms), 2)]
    hbar = terms[0]

    out_ref[0] = jax.lax.dot(
        hbar, w2_ref[...],
        preferred_element_type=jnp.float32) + b2_ref[...]


def kernel(object_features, masks, pos_embed, W1, b1, W2, b2):
    b, k_slots, d = object_features.shape
    p = pos_embed.shape[1]
    h = W1.shape[1]
    out_dim = W2.shape[1] - 1

    sc_topk = functools.partial(
        pl.kernel,
        out_type=jax.ShapeDtypeStruct((b, k_slots * p), jnp.float32),
        mesh=plsc.VectorSubcoreMesh(core_axis_name="c", subcore_axis_name="s"),
        scratch_types=[
            pltpu.VMEM((k_slots * p,), jnp.float32),
            pltpu.VMEM((k_slots * p,), jnp.float32),
        ],
        compiler_params=pltpu.CompilerParams(needs_layout_passes=False),
    )(_sc_topk_kernel)
    selneg = sc_topk(masks.reshape(b, k_slots * p)).reshape(b, k_slots, p)

    x = jnp.concatenate([object_features.reshape(b * k_slots, d), pos_embed[0]], axis=0)
    feat_proj, pos_base, w2_dec, w_alpha = pl.pallas_call(
        _prep_kernel,
        out_shape=(
            jax.ShapeDtypeStruct((b * k_slots, h), jnp.bfloat16),
            jax.ShapeDtypeStruct((p, h), jnp.bfloat16),
            jax.ShapeDtypeStruct((h, out_dim), jnp.bfloat16),
            jax.ShapeDtypeStruct((h, 128), jnp.bfloat16),
        ),
    )(x, W1, b1.reshape(1, h), W2)

    feat_proj = feat_proj.reshape(b, k_slots, h)
    b2_dec = b2[:out_dim].reshape(1, out_dim)

    out = pl.pallas_call(
        _decode_kernel,
        grid=(b,),
        in_specs=[
            pl.BlockSpec((1, k_slots, h), lambda i: (i, 0, 0)),
            pl.BlockSpec((1, k_slots, p), lambda i: (i, 0, 0)),
            pl.BlockSpec((p, h), lambda i: (0, 0)),
            pl.BlockSpec((h, out_dim), lambda i: (0, 0)),
            pl.BlockSpec((h, 128), lambda i: (0, 0)),
            pl.BlockSpec((1, out_dim), lambda i: (0, 0)),
        ],
        out_specs=pl.BlockSpec((1, p, out_dim), lambda i: (i, 0, 0)),
        out_shape=jax.ShapeDtypeStruct((b, p, out_dim), jnp.float32),
    )(feat_proj, selneg, pos_base, w2_dec, w_alpha, b2_dec)
    return out


# speedup vs baseline: 1.0869x; 1.0688x over previous
import functools

import jax
import jax.numpy as jnp
from jax import lax
from jax.experimental import pallas as pl
from jax.experimental.pallas import tpu as pltpu
from jax.experimental.pallas import tpu_sc as plsc

_NEG = -1e30


def _sc_topk_kernel(masks_hbm, out_hbm, in_v, out_v):
    b = lax.axis_index("s") * 2 + lax.axis_index("c")
    pltpu.sync_copy(masks_hbm.at[b], in_v)
    k_slots = 16
    n_pos = out_v.shape[0] // k_slots

    def body(g, carry):
        base = g * 16
        keys = []
        for k in range(k_slots):
            v = in_v[pl.ds(k * n_pos + base, 16)]
            bits = plsc.bitcast(v, jnp.int32)
            keys.append((bits << 4) | (15 - k))
        for k in range(k_slots):
            rank = (keys[0] > keys[k]).astype(jnp.int32)
            for j in range(1, k_slots):
                rank = rank + (keys[j] > keys[k]).astype(jnp.int32)
            vals = jnp.where(rank < 8, 0.0, _NEG).astype(jnp.float32)
            out_v[pl.ds(k * n_pos + base, 16)] = vals
        return carry

    lax.fori_loop(0, n_pos // 16, body, 0)
    pltpu.sync_copy(out_v, out_hbm.at[b])


def _prep_kernel(x_ref, w1_ref, b1_ref, w2_ref, feat_ref, pos_ref,
                 w2d_ref, wa_ref):
    n_feat = feat_ref.shape[0]
    out_dim = w2d_ref.shape[1]
    y = jax.lax.dot(x_ref[...], w1_ref[...], precision=jax.lax.Precision.HIGHEST)
    feat_ref[...] = y[:n_feat].astype(jnp.bfloat16)
    pos_ref[...] = (y[n_feat:] + b1_ref[...]).astype(jnp.bfloat16)
    w2d_ref[...] = (0.5 * w2_ref[:, :out_dim]).astype(jnp.bfloat16)
    wa_ref[...] = (0.5 * w2_ref[:, w2_ref.shape[1] - 128:]).astype(jnp.bfloat16)


def _decode_kernel(feat_ref, sn_ref, pos_ref, w2_ref, wa_ref, b2_ref, out_ref):
    k_slots = feat_ref.shape[1]
    feat = feat_ref[0]
    sn = sn_ref[0]
    pos = pos_ref[...]

    x = feat[:, None, :] + pos[None, :, :]
    c0 = jnp.bfloat16(0.7978845608028654)
    c1 = jnp.bfloat16(0.7978845608028654 * 0.044715)
    t = x * x
    y = x * (c0 + c1 * t)
    h_all = x + x * jnp.tanh(y)

    p_dim, h_dim = pos.shape
    a3 = jax.lax.dot(h_all.reshape(k_slots * p_dim, h_dim), wa_ref[...],
                     preferred_element_type=jnp.float32)
    alpha = a3[:, 127].reshape(k_slots, p_dim)

    a = alpha + sn
    amax = jnp.max(a, axis=0, keepdims=True)
    e = jnp.exp(a - amax)
    w = e / jnp.sum(e, axis=0, keepdims=True)

    wb = w.astype(jnp.bfloat16)
    hw = h_all * wb[:, :, None]
    terms = [hw[k] for k in range(k_slots)]
    while len(terms) > 1:
        terms = [terms[i] + terms[i + 1] for i in range(0, len(terms), 2)]
    hbar = terms[0]

    out_ref[0] = jax.lax.dot(
        hbar, w2_ref[...],
        preferred_element_type=jnp.float32) + b2_ref[...]


def kernel(object_features, masks, pos_embed, W1, b1, W2, b2):
    b, k_slots, d = object_features.shape
    p = pos_embed.shape[1]
    h = W1.shape[1]
    out_dim = W2.shape[1] - 1

    sc_topk = functools.partial(
        pl.kernel,
        out_type=jax.ShapeDtypeStruct((b, k_slots * p), jnp.float32),
        mesh=plsc.VectorSubcoreMesh(core_axis_name="c", subcore_axis_name="s"),
        scratch_types=[
            pltpu.VMEM((k_slots * p,), jnp.float32),
            pltpu.VMEM((k_slots * p,), jnp.float32),
        ],
        compiler_params=pltpu.CompilerParams(needs_layout_passes=False),
    )(_sc_topk_kernel)
    selneg = sc_topk(masks.reshape(b, k_slots * p)).reshape(b, k_slots, p)

    x = jnp.concatenate([object_features.reshape(b * k_slots, d), pos_embed[0]], axis=0)
    feat_proj, pos_base, w2_dec, w_alpha = pl.pallas_call(
        _prep_kernel,
        out_shape=(
            jax.ShapeDtypeStruct((b * k_slots, h), jnp.bfloat16),
            jax.ShapeDtypeStruct((p, h), jnp.bfloat16),
            jax.ShapeDtypeStruct((h, out_dim), jnp.bfloat16),
            jax.ShapeDtypeStruct((h, 128), jnp.bfloat16),
        ),
    )(x, W1, b1.reshape(1, h), W2)

    feat_proj = feat_proj.reshape(b, k_slots, h)
    b2_dec = b2[:out_dim].reshape(1, out_dim)

    out = pl.pallas_call(
        _decode_kernel,
        grid=(b,),
        in_specs=[
            pl.BlockSpec((1, k_slots, h), lambda i: (i, 0, 0)),
            pl.BlockSpec((1, k_slots, p), lambda i: (i, 0, 0)),
            pl.BlockSpec((p, h), lambda i: (0, 0)),
            pl.BlockSpec((h, out_dim), lambda i: (0, 0)),
            pl.BlockSpec((h, 128), lambda i: (0, 0)),
            pl.BlockSpec((1, out_dim), lambda i: (0, 0)),
        ],
        out_specs=pl.BlockSpec((1, p, out_dim), lambda i: (i, 0, 0)),
        out_shape=jax.ShapeDtypeStruct((b, p, out_dim), jnp.float32),
    )(feat_proj, selneg, pos_base, w2_dec, w_alpha, b2_dec)
    return out


# speedup vs baseline: 1.1872x; 1.0923x over previous
import functools

import jax
import jax.numpy as jnp
from jax import lax
from jax.experimental import pallas as pl
from jax.experimental.pallas import tpu as pltpu
from jax.experimental.pallas import tpu_sc as plsc

_NEG = -1e30


def _sc_topk_kernel(masks_hbm, out_hbm, in_v, out_v):
    b = lax.axis_index("s") * 2 + lax.axis_index("c")
    pltpu.sync_copy(masks_hbm.at[b], in_v)
    k_slots = 16
    n_pos = out_v.shape[0] // k_slots

    def body(g, carry):
        base = g * 16
        keys = []
        for k in range(k_slots):
            v = in_v[pl.ds(k * n_pos + base, 16)]
            bits = plsc.bitcast(v, jnp.int32)
            keys.append((bits << 4) | (15 - k))
        for k in range(k_slots):
            rank = (keys[0] > keys[k]).astype(jnp.int32)
            for j in range(1, k_slots):
                rank = rank + (keys[j] > keys[k]).astype(jnp.int32)
            vals = jnp.where(rank < 8, 0.0, _NEG).astype(jnp.float32)
            out_v[pl.ds(k * n_pos + base, 16)] = vals
        return carry

    lax.fori_loop(0, n_pos // 16, body, 0)
    pltpu.sync_copy(out_v, out_hbm.at[b])


def _prep_kernel(x_ref, w1_ref, b1_ref, w2_ref, feat_ref, pos_ref,
                 w2d_ref, wa_ref):
    n_feat = feat_ref.shape[0]
    out_dim = w2d_ref.shape[1]
    y = jax.lax.dot(x_ref[...], w1_ref[...], precision=jax.lax.Precision.HIGHEST)
    feat_ref[...] = y[:n_feat].astype(jnp.bfloat16)
    pos_ref[...] = (y[n_feat:] + b1_ref[...]).astype(jnp.bfloat16)
    w2d_ref[...] = (0.5 * w2_ref[:, :out_dim]).astype(jnp.bfloat16)
    wa_col = 0.5 * w2_ref[:, out_dim:out_dim + 1]
    wa_ref[...] = jnp.concatenate(
        [wa_col, jnp.zeros((wa_col.shape[0], 127), jnp.float32)],
        axis=1).astype(jnp.bfloat16)


def _decode_kernel(feat_ref, sn_ref, pos_ref, w2_ref, wa_ref, b2_ref, out_ref):
    k_slots = feat_ref.shape[1]
    feat = feat_ref[0]
    sn = sn_ref[0]
    pos = pos_ref[...]

    p_dim, h_dim = pos.shape
    c0 = jnp.bfloat16(0.7978845608028654)
    c1 = jnp.bfloat16(0.7978845608028654 * 0.044715)
    half = k_slots // 2
    h_halves, alpha_halves = [], []
    for lo in (0, half):
        x = feat[lo:lo + half, None, :] + pos[None, :, :]
        t = x * x
        y = x * (c0 + c1 * t)
        h_half = x + x * jnp.tanh(y)
        a3 = jax.lax.dot(h_half.reshape(half * p_dim, h_dim), wa_ref[...],
                         preferred_element_type=jnp.float32)
        h_halves.append(h_half)
        alpha_halves.append(a3[:, 0].reshape(half, p_dim))
    alpha = jnp.concatenate(alpha_halves, axis=0)

    a = alpha + sn
    amax = jnp.max(a, axis=0, keepdims=True)
    e = jnp.exp(a - amax)
    w = e / jnp.sum(e, axis=0, keepdims=True)

    w3 = w[:, :, None]
    hw_halves = [h_halves[g] * w3[g * half:(g + 1) * half].astype(jnp.bfloat16)
                 for g in range(2)]
    terms = [hw_halves[g][k] for g in range(2) for k in range(half)]
    while len(terms) > 1:
        terms = [terms[i] + terms[i + 1] for i in range(0, len(terms), 2)]
    out_ref[0] = jax.lax.dot(
        terms[0], w2_ref[...],
        preferred_element_type=jnp.float32) + b2_ref[...]


def kernel(object_features, masks, pos_embed, W1, b1, W2, b2):
    b, k_slots, d = object_features.shape
    p = pos_embed.shape[1]
    h = W1.shape[1]
    out_dim = W2.shape[1] - 1

    sc_topk = functools.partial(
        pl.kernel,
        out_type=jax.ShapeDtypeStruct((b, k_slots * p), jnp.float32),
        mesh=plsc.VectorSubcoreMesh(core_axis_name="c", subcore_axis_name="s"),
        scratch_types=[
            pltpu.VMEM((k_slots * p,), jnp.float32),
            pltpu.VMEM((k_slots * p,), jnp.float32),
        ],
        compiler_params=pltpu.CompilerParams(needs_layout_passes=False),
    )(_sc_topk_kernel)
    selneg = sc_topk(masks.reshape(b, k_slots * p)).reshape(b, k_slots, p)

    x = jnp.concatenate([object_features.reshape(b * k_slots, d), pos_embed[0]], axis=0)
    feat_proj, pos_base, w2_dec, w_alpha = pl.pallas_call(
        _prep_kernel,
        out_shape=(
            jax.ShapeDtypeStruct((b * k_slots, h), jnp.bfloat16),
            jax.ShapeDtypeStruct((p, h), jnp.bfloat16),
            jax.ShapeDtypeStruct((h, out_dim), jnp.bfloat16),
            jax.ShapeDtypeStruct((h, 128), jnp.bfloat16),
        ),
    )(x, W1, b1.reshape(1, h), W2)

    feat_proj = feat_proj.reshape(b, k_slots, h)
    b2_dec = b2[:out_dim].reshape(1, out_dim)

    out = pl.pallas_call(
        _decode_kernel,
        grid=(b,),
        in_specs=[
            pl.BlockSpec((1, k_slots, h), lambda i: (i, 0, 0)),
            pl.BlockSpec((1, k_slots, p), lambda i: (i, 0, 0)),
            pl.BlockSpec((p, h), lambda i: (0, 0)),
            pl.BlockSpec((h, out_dim), lambda i: (0, 0)),
            pl.BlockSpec((h, 128), lambda i: (0, 0)),
            pl.BlockSpec((1, out_dim), lambda i: (0, 0)),
        ],
        out_specs=pl.BlockSpec((1, p, out_dim), lambda i: (i, 0, 0)),
        out_shape=jax.ShapeDtypeStruct((b, p, out_dim), jnp.float32),
    )(feat_proj, selneg, pos_base, w2_dec, w_alpha, b2_dec)
    return out


# speedup vs baseline: 1.2811x; 1.0791x over previous
import functools

import jax
import jax.numpy as jnp
from jax import lax
from jax.experimental import pallas as pl
from jax.experimental.pallas import tpu as pltpu
from jax.experimental.pallas import tpu_sc as plsc

_NEG = -1e30


def _sc_topk_kernel(masks_hbm, out_hbm, in_v, out_v):
    b = lax.axis_index("s") * 2 + lax.axis_index("c")
    pltpu.sync_copy(masks_hbm.at[b], in_v)
    k_slots = 16
    n_pos = out_v.shape[0] // k_slots

    def body(g, carry):
        base = g * 16
        keys = []
        for k in range(k_slots):
            v = in_v[pl.ds(k * n_pos + base, 16)]
            bits = plsc.bitcast(v, jnp.int32)
            keys.append((bits << 4) | (15 - k))
        for k in range(k_slots):
            rank = (keys[0] > keys[k]).astype(jnp.int32)
            for j in range(1, k_slots):
                rank = rank + (keys[j] > keys[k]).astype(jnp.int32)
            vals = jnp.where(rank < 8, 0.0, _NEG).astype(jnp.float32)
            out_v[pl.ds(k * n_pos + base, 16)] = vals
        return carry

    lax.fori_loop(0, n_pos // 16, body, 0)
    pltpu.sync_copy(out_v, out_hbm.at[b])


def _prep_kernel(x_ref, w1_ref, b1_ref, w2_ref, feat_ref, pos_ref,
                 w2d_ref, wa_ref):
    n_feat = feat_ref.shape[0]
    out_dim = w2d_ref.shape[1]
    y = jax.lax.dot(x_ref[...], w1_ref[...], precision=jax.lax.Precision.HIGHEST)
    feat_ref[...] = y[:n_feat].astype(jnp.bfloat16)
    pos_ref[...] = (y[n_feat:] + b1_ref[...]).astype(jnp.bfloat16)
    w2d_ref[...] = (0.5 * w2_ref[:, :out_dim]).astype(jnp.bfloat16)
    wa_col = 0.5 * w2_ref[:, out_dim:out_dim + 1]
    wa_ref[...] = jnp.concatenate(
        [wa_col, jnp.zeros((wa_col.shape[0], 127), jnp.float32)],
        axis=1).astype(jnp.bfloat16)


def _decode_kernel(feat_ref, sn_ref, pos_ref, w2_ref, wa_ref, b2_ref, out_ref):
    k_slots = feat_ref.shape[1]
    feat = feat_ref[0]
    sn = sn_ref[0]
    pos = pos_ref[...]

    p_dim, h_dim = pos.shape
    c0 = jnp.bfloat16(0.7978845608028654)
    c1 = jnp.bfloat16(0.7978845608028654 * 0.044715)
    n_chunks = 4
    half = k_slots // n_chunks
    h_halves, alpha_halves = [], []
    for lo in range(0, k_slots, half):
        x = feat[lo:lo + half, None, :] + pos[None, :, :]
        t = x * x
        y = x * (c0 + c1 * t)
        h_half = x + x * jnp.tanh(y)
        a3 = jax.lax.dot(h_half.reshape(half * p_dim, h_dim), wa_ref[...],
                         preferred_element_type=jnp.float32)
        h_halves.append(h_half)
        alpha_halves.append(a3[:, 0].reshape(half, p_dim))
    alpha = jnp.concatenate(alpha_halves, axis=0)

    a = alpha + sn
    amax = jnp.max(a, axis=0, keepdims=True)
    e = jnp.exp(a - amax)
    w = e / jnp.sum(e, axis=0, keepdims=True)

    w3 = w[:, :, None]
    hw_halves = [h_halves[g] * w3[g * half:(g + 1) * half].astype(jnp.bfloat16)
                 for g in range(n_chunks)]
    p_half = p_dim // 2
    for pi in range(2):
        p0, p1 = pi * p_half, (pi + 1) * p_half
        terms = [hw_halves[g][k, p0:p1, :]
                 for g in range(n_chunks) for k in range(half)]
        while len(terms) > 1:
            terms = [terms[i] + terms[i + 1] for i in range(0, len(terms), 2)]
        out_ref[0, p0:p1, :] = jax.lax.dot(
            terms[0], w2_ref[...],
            preferred_element_type=jnp.float32) + b2_ref[...]


def kernel(object_features, masks, pos_embed, W1, b1, W2, b2):
    b, k_slots, d = object_features.shape
    p = pos_embed.shape[1]
    h = W1.shape[1]
    out_dim = W2.shape[1] - 1

    sc_topk = functools.partial(
        pl.kernel,
        out_type=jax.ShapeDtypeStruct((b, k_slots * p), jnp.float32),
        mesh=plsc.VectorSubcoreMesh(core_axis_name="c", subcore_axis_name="s"),
        scratch_types=[
            pltpu.VMEM((k_slots * p,), jnp.float32),
            pltpu.VMEM((k_slots * p,), jnp.float32),
        ],
        compiler_params=pltpu.CompilerParams(needs_layout_passes=False),
    )(_sc_topk_kernel)
    selneg = sc_topk(masks.reshape(b, k_slots * p)).reshape(b, k_slots, p)

    x = jnp.concatenate([object_features.reshape(b * k_slots, d), pos_embed[0]], axis=0)
    feat_proj, pos_base, w2_dec, w_alpha = pl.pallas_call(
        _prep_kernel,
        out_shape=(
            jax.ShapeDtypeStruct((b * k_slots, h), jnp.bfloat16),
            jax.ShapeDtypeStruct((p, h), jnp.bfloat16),
            jax.ShapeDtypeStruct((h, out_dim), jnp.bfloat16),
            jax.ShapeDtypeStruct((h, 128), jnp.bfloat16),
        ),
    )(x, W1, b1.reshape(1, h), W2)

    feat_proj = feat_proj.reshape(b, k_slots, h)
    b2_dec = b2[:out_dim].reshape(1, out_dim)

    out = pl.pallas_call(
        _decode_kernel,
        grid=(b,),
        in_specs=[
            pl.BlockSpec((1, k_slots, h), lambda i: (i, 0, 0)),
            pl.BlockSpec((1, k_slots, p), lambda i: (i, 0, 0)),
            pl.BlockSpec((p, h), lambda i: (0, 0)),
            pl.BlockSpec((h, out_dim), lambda i: (0, 0)),
            pl.BlockSpec((h, 128), lambda i: (0, 0)),
            pl.BlockSpec((1, out_dim), lambda i: (0, 0)),
        ],
        out_specs=pl.BlockSpec((1, p, out_dim), lambda i: (i, 0, 0)),
        out_shape=jax.ShapeDtypeStruct((b, p, out_dim), jnp.float32),
    )(feat_proj, selneg, pos_base, w2_dec, w_alpha, b2_dec)
    return out
